# baseline (speedup 1.0000x reference)
import jax
import jax.numpy as jnp
from jax.experimental import pallas as pl
from jax.experimental.pallas import tpu as pltpu

_TILE = 512
_K = 1024
_D = 256
_NB = 8


def _rvq_body(xt_ref, cb_ref, cbt_ref, n_ref, q_ref, codes_ref):
    r = xt_ref[...]
    q = jnp.zeros_like(r)
    iota = jax.lax.broadcasted_iota(jnp.int32, (_TILE, _K), 1).astype(jnp.float32)
    iota_row = jax.lax.broadcasted_iota(jnp.int32, (1, _K), 1).astype(jnp.float32)
    for i in range(_NB):
        sumx = jnp.sum(r * r, axis=1, keepdims=True)
        mm = jax.lax.dot_general(
            r, cbt_ref[i],
            (((1,), (0,)), ((), ())),
            preferred_element_type=jnp.float32,
        )
        d2 = (sumx - 2.0 * mm) + n_ref[i][None, :]
        dist = jnp.sqrt(jnp.maximum(d2, 0.0))
        m = jnp.min(dist, axis=1, keepdims=True)
        first = jnp.min(jnp.where(dist == m, iota, float(_K)), axis=1,
                        keepdims=True)
        onehot = (iota == first).astype(jnp.float32)
        qr = jax.lax.dot_general(
            onehot, cb_ref[i],
            (((1,), (0,)), ((), ())),
            precision=jax.lax.Precision.HIGHEST,
            preferred_element_type=jnp.float32,
        )
        codes_row = jax.lax.dot_general(
            iota_row, onehot,
            (((1,), (1,)), ((), ())),
            precision=jax.lax.Precision.HIGHEST,
            preferred_element_type=jnp.float32,
        )
        codes_ref[i:i + 1, :] = codes_row.astype(jnp.int32)
        r = r - qr
        q = q + qr
    q_ref[...] = q


def kernel(x, codebooks):
    B, D, T = x.shape
    NB, K, _ = codebooks.shape
    N = B * T
    xt = jnp.transpose(x, (0, 2, 1)).reshape(N, D)
    cbt = jnp.transpose(codebooks, (0, 2, 1))
    norms = jnp.sum(codebooks * codebooks, axis=-1)

    q_flat, codes_flat = pl.pallas_call(
        _rvq_body,
        grid=(N // _TILE,),
        in_specs=[
            pl.BlockSpec((_TILE, D), lambda t: (t, 0)),
            pl.BlockSpec((NB, K, D), lambda t: (0, 0, 0)),
            pl.BlockSpec((NB, D, K), lambda t: (0, 0, 0)),
            pl.BlockSpec((NB, K), lambda t: (0, 0)),
        ],
        out_specs=[
            pl.BlockSpec((_TILE, D), lambda t: (t, 0)),
            pl.BlockSpec((NB, _TILE), lambda t: (0, t)),
        ],
        out_shape=[
            jax.ShapeDtypeStruct((N, D), jnp.float32),
            jax.ShapeDtypeStruct((NB, N), jnp.int32),
        ],
        compiler_params=pltpu.CompilerParams(
            dimension_semantics=("parallel",),
        ),
    )(xt, codebooks, cbt, norms)

    quantized = q_flat.reshape(B, T, D).transpose(0, 2, 1)
    codes = codes_flat.reshape(NB, B, T).transpose(1, 0, 2)
    return quantized, codes


# speedup vs baseline: 1.0083x; 1.0083x over previous
import jax
import jax.numpy as jnp
from jax.experimental import pallas as pl
from jax.experimental.pallas import tpu as pltpu

_TILE = 512
_K = 1024
_D = 256
_NB = 8


def _rvq_body(xt_ref, cb_ref, cbt_ref, n_ref, ident_ref, q_ref, codes_ref):
    x0 = xt_ref[...]
    r = x0
    q = jnp.zeros_like(r)
    f = jnp.zeros((_TILE, _NB), jnp.float32)
    iota = jax.lax.broadcasted_iota(jnp.int32, (_TILE, _K), 1)
    iota8 = jax.lax.broadcasted_iota(jnp.int32, (_TILE, _NB), 1)
    for i in range(_NB):
        sumx = jnp.sum(r * r, axis=1, keepdims=True)
        mm = jax.lax.dot_general(
            r, cbt_ref[i],
            (((1,), (0,)), ((), ())),
            preferred_element_type=jnp.float32,
        )
        d2 = (sumx - 2.0 * mm) + n_ref[i][None, :]
        dist = jnp.sqrt(jnp.maximum(d2, 0.0))
        idx = jnp.argmin(dist, axis=1).reshape(_TILE, 1)
        onehot = (iota == idx).astype(jnp.float32)
        qr = jax.lax.dot_general(
            onehot, cb_ref[i],
            (((1,), (0,)), ((), ())),
            precision=jax.lax.Precision.HIGHEST,
            preferred_element_type=jnp.float32,
        )
        f = f + jnp.where(iota8 == i, idx.astype(jnp.float32), 0.0)
        r = r - qr
        q = q + qr
    codes_all = jax.lax.dot_general(
        f, ident_ref[...],
        (((0,), (0,)), ((), ())),
        precision=jax.lax.Precision.HIGHEST,
        preferred_element_type=jnp.float32,
    )
    codes_ref[...] = codes_all.astype(jnp.int32)
    q_ref[...] = x0 + (q - x0)


def kernel(x, codebooks):
    B, D, T = x.shape
    NB, K, _ = codebooks.shape
    N = B * T
    xt = jnp.transpose(x, (0, 2, 1)).reshape(N, D)
    cbt = jnp.transpose(codebooks, (0, 2, 1))
    norms = jnp.sum(codebooks * codebooks, axis=-1)
    ident = jnp.eye(_TILE, dtype=jnp.float32)

    q_flat, codes_flat = pl.pallas_call(
        _rvq_body,
        grid=(N // _TILE,),
        in_specs=[
            pl.BlockSpec((_TILE, D), lambda t: (t, 0)),
            pl.BlockSpec((NB, K, D), lambda t: (0, 0, 0)),
            pl.BlockSpec((NB, D, K), lambda t: (0, 0, 0)),
            pl.BlockSpec((NB, K), lambda t: (0, 0)),
            pl.BlockSpec((_TILE, _TILE), lambda t: (0, 0)),
        ],
        out_specs=[
            pl.BlockSpec((_TILE, D), lambda t: (t, 0)),
            pl.BlockSpec((NB, _TILE), lambda t: (0, t)),
        ],
        out_shape=[
            jax.ShapeDtypeStruct((N, D), jnp.float32),
            jax.ShapeDtypeStruct((NB, N), jnp.int32),
        ],
        compiler_params=pltpu.CompilerParams(
            dimension_semantics=("parallel",),
        ),
    )(xt, codebooks, cbt, norms, ident)

    quantized = q_flat.reshape(B, T, D).transpose(0, 2, 1)
    codes = codes_flat.reshape(NB, B, T).transpose(1, 0, 2)
    return quantized, codes


# speedup vs baseline: 1.5665x; 1.5535x over previous
import jax
import jax.numpy as jnp
from jax.experimental import pallas as pl
from jax.experimental.pallas import tpu as pltpu

_TILE = 512
_HALF = _TILE // 2
_K = 1024
_D = 256
_NB = 8


def _rvq_body(xt_ref, cb_ref, cbt_ref, n_ref, ident_ref, q_ref, codes_ref):
    x0 = xt_ref[...]
    rs = [x0[:_HALF], x0[_HALF:]]
    qs = [jnp.zeros_like(rs[0]), jnp.zeros_like(rs[1])]
    fs = [jnp.zeros((_HALF, _NB), jnp.float32),
          jnp.zeros((_HALF, _NB), jnp.float32)]
    iota = jax.lax.broadcasted_iota(jnp.int32, (_HALF, _K), 1).astype(jnp.float32)
    iota8 = jax.lax.broadcasted_iota(jnp.int32, (_HALF, _NB), 1)
    for i in range(_NB):
        for s in range(2):
            r = rs[s]
            sumx = jnp.sum(r * r, axis=1, keepdims=True)
            mm = jax.lax.dot_general(
                r, cbt_ref[i],
                (((1,), (0,)), ((), ())),
                preferred_element_type=jnp.float32,
            )
            d2 = (sumx - 2.0 * mm) + n_ref[i][None, :]
            dist = jnp.sqrt(jnp.maximum(d2, 0.0))
            m = jnp.min(dist, axis=1, keepdims=True)
            first = jnp.min(jnp.where(dist == m, iota, float(_K)), axis=1,
                            keepdims=True)
            onehot = (iota == first).astype(jnp.float32)
            qr = jax.lax.dot_general(
                onehot, cb_ref[i],
                (((1,), (0,)), ((), ())),
                precision=jax.lax.Precision.HIGHEST,
                preferred_element_type=jnp.float32,
            )
            fs[s] = fs[s] + jnp.where(iota8 == i, first, 0.0)
            rs[s] = r - qr
            qs[s] = qs[s] + qr
    for s in range(2):
        codes_half = jax.lax.dot_general(
            fs[s], ident_ref[...],
            (((0,), (0,)), ((), ())),
            precision=jax.lax.Precision.HIGHEST,
            preferred_element_type=jnp.float32,
        )
        codes_ref[:, s * _HALF:(s + 1) * _HALF] = codes_half.astype(jnp.int32)
    q_ref[...] = x0 + (jnp.concatenate([qs[0], qs[1]], axis=0) - x0)


def kernel(x, codebooks):
    B, D, T = x.shape
    NB, K, _ = codebooks.shape
    N = B * T
    xt = jnp.transpose(x, (0, 2, 1)).reshape(N, D)
    cbt = jnp.transpose(codebooks, (0, 2, 1))
    norms = jnp.sum(codebooks * codebooks, axis=-1)
    ident = jnp.eye(_HALF, dtype=jnp.float32)

    q_flat, codes_flat = pl.pallas_call(
        _rvq_body,
        grid=(N // _TILE,),
        in_specs=[
            pl.BlockSpec((_TILE, D), lambda t: (t, 0)),
            pl.BlockSpec((NB, K, D), lambda t: (0, 0, 0)),
            pl.BlockSpec((NB, D, K), lambda t: (0, 0, 0)),
            pl.BlockSpec((NB, K), lambda t: (0, 0)),
            pl.BlockSpec((_HALF, _HALF), lambda t: (0, 0)),
        ],
        out_specs=[
            pl.BlockSpec((_TILE, D), lambda t: (t, 0)),
            pl.BlockSpec((NB, _TILE), lambda t: (0, t)),
        ],
        out_shape=[
            jax.ShapeDtypeStruct((N, D), jnp.float32),
            jax.ShapeDtypeStruct((NB, N), jnp.int32),
        ],
        compiler_params=pltpu.CompilerParams(
            dimension_semantics=("parallel",),
        ),
    )(xt, codebooks, cbt, norms, ident)

    quantized = q_flat.reshape(B, T, D).transpose(0, 2, 1)
    codes = codes_flat.reshape(NB, B, T).transpose(1, 0, 2)
    return quantized, codes
